# baseline (speedup 1.0000x reference)
import functools

import jax
import jax.numpy as jnp
from jax import lax
from jax.experimental import pallas as pl
from jax.experimental.pallas import tpu as pltpu
from jax.experimental.pallas import tpu_sc as plsc

_B, _S, _D, _E, _F = 4, 2048, 1024, 16, 4096
_T = _B * _S
_RB = 1024
_NRB = _T // _RB
_SG = 2048
_SUB = 512
_NSUB = _SG // _SUB
_TKP = 2 * _T + _E * _SG
_NT = _TKP // _SG
_BF = 512
_NF = _F // _BF

_NW = 32
_ROWS_W = _T // _NW
_CH = 64
_NCH = _ROWS_W // _CH


def _router_kernel(x_ref, gw_ref, ints_ref, flts_ref, cnt_out_ref, cnt_ref):
    i = pl.program_id(0)

    @pl.when(i == 0)
    def _():
        cnt_ref[...] = jnp.zeros_like(cnt_ref)

    x = x_ref[...]
    logits = jnp.dot(x, gw_ref[...], preferred_element_type=jnp.float32)
    eidx = lax.broadcasted_iota(jnp.int32, logits.shape, 1)
    m0 = jnp.max(logits, axis=-1, keepdims=True)
    i0 = jnp.min(jnp.where(logits == m0, eidx, _E), axis=-1, keepdims=True)
    l2 = jnp.where(eidx == i0, -jnp.inf, logits)
    m1 = jnp.max(l2, axis=-1, keepdims=True)
    i1 = jnp.min(jnp.where(l2 == m1, eidx, _E), axis=-1, keepdims=True)
    a = jnp.exp(m1 - m0)
    w0 = 1.0 / (1.0 + a)
    w1 = a / (1.0 + a)

    oh0 = (eidx == i0).astype(jnp.float32)
    oh1 = (eidx == i1).astype(jnp.float32)
    ohs = oh0 + oh1
    r = lax.broadcasted_iota(jnp.int32, (_RB, _RB), 0)
    c = lax.broadcasted_iota(jnp.int32, (_RB, _RB), 1)
    tri = (r > c).astype(jnp.float32)
    prefix = jnp.dot(tri, ohs, preferred_element_type=jnp.float32)
    base = cnt_ref[0:1, 0:_E]
    tot = prefix + base
    pos0 = jnp.sum(tot * oh0, axis=-1, keepdims=True)
    pos1 = jnp.sum(tot * oh1, axis=-1, keepdims=True)

    ints_ref[:, 0:4] = jnp.concatenate(
        [i0, i1, pos0.astype(jnp.int32), pos1.astype(jnp.int32)], axis=1)
    flts_ref[:, 0:2] = jnp.concatenate([w0, w1], axis=1)
    newcnt = base + jnp.sum(ohs, axis=0, keepdims=True)
    cnt_ref[0:1, 0:_E] = newcnt
    cnt_out_ref[0:1, 0:_E] = newcnt.astype(jnp.int32)


def _run_router(x, gate_w):
    return pl.pallas_call(
        _router_kernel,
        grid=(_NRB,),
        in_specs=[
            pl.BlockSpec((_RB, _D), lambda i: (i, 0)),
            pl.BlockSpec((_D, _E), lambda i: (0, 0)),
        ],
        out_specs=[
            pl.BlockSpec((_RB, 128), lambda i: (i, 0)),
            pl.BlockSpec((_RB, 128), lambda i: (i, 0)),
            pl.BlockSpec((8, 128), lambda i: (0, 0)),
        ],
        out_shape=[
            jax.ShapeDtypeStruct((_T, 128), jnp.int32),
            jax.ShapeDtypeStruct((_T, 128), jnp.float32),
            jax.ShapeDtypeStruct((8, 128), jnp.int32),
        ],
        scratch_shapes=[pltpu.VMEM((8, 128), jnp.float32)],
    )(x, gate_w)


def _dispatch_kernel(x_hbm, s0_hbm, s1_hbm, xs_hbm, xv, i0v, i1v, sem):
    wid = lax.axis_index("s") * 2 + lax.axis_index("c")
    pltpu.sync_copy(s0_hbm.at[wid], i0v)
    pltpu.sync_copy(s1_hbm.at[wid], i1v)
    for j in range(_NCH):
        base = wid * _ROWS_W + j * _CH
        pltpu.sync_copy(x_hbm.at[pl.ds(base, _CH), :], xv)
        pltpu.async_copy(xv, xs_hbm.at[i0v.at[j]], sem).wait()
        pltpu.async_copy(xv, xs_hbm.at[i1v.at[j]], sem).wait()


def _run_dispatch(x, s0r, s1r):
    mesh = plsc.VectorSubcoreMesh(core_axis_name="c", subcore_axis_name="s")
    return pl.kernel(
        _dispatch_kernel,
        out_type=jax.ShapeDtypeStruct((_TKP, _D), jnp.float32),
        mesh=mesh,
        scratch_types=[
            pltpu.VMEM((_CH, _D), jnp.float32),
            pltpu.VMEM((_NCH, _CH), jnp.int32),
            pltpu.VMEM((_NCH, _CH), jnp.int32),
            pltpu.SemaphoreType.DMA,
        ],
    )(x, s0r, s1r)


def _gmm_kernel(e_ref, u_ref, xs_ref, wg_ref, wu_ref, wd_ref, ys_ref):
    f = pl.program_id(1)
    used = u_ref[pl.program_id(0)]
    wg = wg_ref[0]
    wu = wu_ref[0]
    wd = wd_ref[0]
    for j in range(_NSUB):
        @pl.when(j * _SUB < used)
        def _(j=j):
            sl = pl.ds(j * _SUB, _SUB)
            xsub = xs_ref[sl, :]
            g = jnp.dot(xsub, wg, preferred_element_type=jnp.float32)
            u = jnp.dot(xsub, wu, preferred_element_type=jnp.float32)
            h = g * lax.logistic(g) * u
            y = jnp.dot(h, wd, preferred_element_type=jnp.float32)

            @pl.when(f == 0)
            def _():
                ys_ref[sl, :] = y

            @pl.when(f > 0)
            def _():
                ys_ref[sl, :] += y


def _run_gmm(xs, w_gate, w_up, w_down, e_of_tile, used):
    grid_spec = pltpu.PrefetchScalarGridSpec(
        num_scalar_prefetch=2,
        grid=(_NT, _NF),
        in_specs=[
            pl.BlockSpec((_SG, _D),
                         lambda i, f, e, u: (jnp.where(u[i] > 0, i, 0), 0)),
            pl.BlockSpec((1, _D, _BF),
                         lambda i, f, e, u: (e[i], 0, jnp.where(u[i] > 0, f, 0))),
            pl.BlockSpec((1, _D, _BF),
                         lambda i, f, e, u: (e[i], 0, jnp.where(u[i] > 0, f, 0))),
            pl.BlockSpec((1, _BF, _D),
                         lambda i, f, e, u: (e[i], jnp.where(u[i] > 0, f, 0), 0)),
        ],
        out_specs=pl.BlockSpec((_SG, _D), lambda i, f, e, u: (i, 0)),
    )
    return pl.pallas_call(
        _gmm_kernel,
        grid_spec=grid_spec,
        out_shape=jax.ShapeDtypeStruct((_TKP, _D), jnp.float32),
    )(e_of_tile, used, xs, w_gate, w_up, w_down)


def _collect_kernel(ys_hbm, s0_hbm, s1_hbm, y0_hbm, y1_hbm, rv, i0v, i1v, sem):
    wid = lax.axis_index("s") * 2 + lax.axis_index("c")
    pltpu.sync_copy(s0_hbm.at[wid], i0v)
    pltpu.sync_copy(s1_hbm.at[wid], i1v)
    for j in range(_NCH):
        base = wid * _ROWS_W + j * _CH
        pltpu.async_copy(ys_hbm.at[i0v.at[j]], rv, sem).wait()
        pltpu.sync_copy(rv, y0_hbm.at[pl.ds(base, _CH), :])
        pltpu.async_copy(ys_hbm.at[i1v.at[j]], rv, sem).wait()
        pltpu.sync_copy(rv, y1_hbm.at[pl.ds(base, _CH), :])


def _run_collect(ys, s0r, s1r):
    mesh = plsc.VectorSubcoreMesh(core_axis_name="c", subcore_axis_name="s")
    return pl.kernel(
        _collect_kernel,
        out_type=(
            jax.ShapeDtypeStruct((_T, _D), jnp.float32),
            jax.ShapeDtypeStruct((_T, _D), jnp.float32),
        ),
        mesh=mesh,
        scratch_types=[
            pltpu.VMEM((_CH, _D), jnp.float32),
            pltpu.VMEM((_NCH, _CH), jnp.int32),
            pltpu.VMEM((_NCH, _CH), jnp.int32),
            pltpu.SemaphoreType.DMA,
        ],
    )(ys, s0r, s1r)


def _combine_kernel(y0_ref, y1_ref, w_ref, out_ref):
    w = w_ref[...]
    out_ref[...] = y0_ref[...] * w[:, 0:1] + y1_ref[...] * w[:, 1:2]


def _run_combine(y0, y1, flts):
    return pl.pallas_call(
        _combine_kernel,
        grid=(_NRB,),
        in_specs=[
            pl.BlockSpec((_RB, _D), lambda i: (i, 0)),
            pl.BlockSpec((_RB, _D), lambda i: (i, 0)),
            pl.BlockSpec((_RB, 128), lambda i: (i, 0)),
        ],
        out_specs=pl.BlockSpec((_RB, _D), lambda i: (i, 0)),
        out_shape=jax.ShapeDtypeStruct((_T, _D), jnp.float32),
    )(y0, y1, flts)


def kernel(hidden_states, gate_w, w_gate, w_up, w_down):
    b, s, d = hidden_states.shape
    x = hidden_states.reshape(-1, d)

    ints, flts, cnts = _run_router(x, gate_w)
    i0 = ints[:, 0]
    i1 = ints[:, 1]
    p0 = ints[:, 2]
    p1 = ints[:, 3]
    counts = cnts[0, :_E]

    sizes_pad = ((counts + _SG - 1) // _SG) * _SG
    ends = jnp.cumsum(sizes_pad)
    offs = ends - sizes_pad
    slot0 = (offs[i0] + p0).astype(jnp.int32)
    slot1 = (offs[i1] + p1).astype(jnp.int32)
    s0r = slot0.reshape(_NW, _NCH, _CH)
    s1r = slot1.reshape(_NW, _NCH, _CH)
    tile_start = jnp.arange(_NT, dtype=jnp.int32) * _SG
    e_of_tile = jnp.minimum(
        jnp.searchsorted(ends, tile_start, side="right"), _E - 1
    ).astype(jnp.int32)
    used = jnp.clip(counts[e_of_tile] - (tile_start - offs[e_of_tile]),
                    0, _SG).astype(jnp.int32)

    xs = _run_dispatch(x, s0r, s1r)
    ys = _run_gmm(xs, w_gate, w_up, w_down, e_of_tile, used)
    y0, y1 = _run_collect(ys, s0r, s1r)
    out = _run_combine(y0, y1, flts)
    return out.reshape(b, s, d)


# speedup vs baseline: 1.1229x; 1.1229x over previous
import functools

import jax
import jax.numpy as jnp
from jax import lax
from jax.experimental import pallas as pl
from jax.experimental.pallas import tpu as pltpu
from jax.experimental.pallas import tpu_sc as plsc

_B, _S, _D, _E, _F = 4, 2048, 1024, 16, 4096
_T = _B * _S
_RB = 1024
_NRB = _T // _RB
_SG = 2048
_SUB = 512
_NSUB = _SG // _SUB
_TKP = 2 * _T + _E * _SG
_NT = _TKP // _SG
_BF = 512
_NF = _F // _BF

_NW = 32
_ROWS_W = _T // _NW
_CH = 64
_NCH = _ROWS_W // _CH


def _router_kernel(x_ref, gw_ref, ints_ref, flts_ref, cnt_out_ref, cnt_ref):
    i = pl.program_id(0)

    @pl.when(i == 0)
    def _():
        cnt_ref[...] = jnp.zeros_like(cnt_ref)

    x = x_ref[...]
    logits = jnp.dot(x, gw_ref[...], preferred_element_type=jnp.float32)
    eidx = lax.broadcasted_iota(jnp.int32, logits.shape, 1)
    m0 = jnp.max(logits, axis=-1, keepdims=True)
    i0 = jnp.min(jnp.where(logits == m0, eidx, _E), axis=-1, keepdims=True)
    l2 = jnp.where(eidx == i0, -jnp.inf, logits)
    m1 = jnp.max(l2, axis=-1, keepdims=True)
    i1 = jnp.min(jnp.where(l2 == m1, eidx, _E), axis=-1, keepdims=True)
    a = jnp.exp(m1 - m0)
    w0 = 1.0 / (1.0 + a)
    w1 = a / (1.0 + a)

    oh0 = (eidx == i0).astype(jnp.float32)
    oh1 = (eidx == i1).astype(jnp.float32)
    ohs = oh0 + oh1
    r = lax.broadcasted_iota(jnp.int32, (_RB, _RB), 0)
    c = lax.broadcasted_iota(jnp.int32, (_RB, _RB), 1)
    tri = (r > c).astype(jnp.float32)
    prefix = jnp.dot(tri, ohs, preferred_element_type=jnp.float32)
    base = cnt_ref[0:1, 0:_E]
    tot = prefix + base
    pos0 = jnp.sum(tot * oh0, axis=-1, keepdims=True)
    pos1 = jnp.sum(tot * oh1, axis=-1, keepdims=True)

    ints_ref[:, 0:4] = jnp.concatenate(
        [i0, i1, pos0.astype(jnp.int32), pos1.astype(jnp.int32)], axis=1)
    flts_ref[:, 0:2] = jnp.concatenate([w0, w1], axis=1)
    newcnt = base + jnp.sum(ohs, axis=0, keepdims=True)
    cnt_ref[0:1, 0:_E] = newcnt
    cnt_out_ref[0:1, 0:_E] = newcnt.astype(jnp.int32)


def _run_router(x, gate_w):
    return pl.pallas_call(
        _router_kernel,
        grid=(_NRB,),
        in_specs=[
            pl.BlockSpec((_RB, _D), lambda i: (i, 0)),
            pl.BlockSpec((_D, _E), lambda i: (0, 0)),
        ],
        out_specs=[
            pl.BlockSpec((_RB, 128), lambda i: (i, 0)),
            pl.BlockSpec((_RB, 128), lambda i: (i, 0)),
            pl.BlockSpec((8, 128), lambda i: (0, 0)),
        ],
        out_shape=[
            jax.ShapeDtypeStruct((_T, 128), jnp.int32),
            jax.ShapeDtypeStruct((_T, 128), jnp.float32),
            jax.ShapeDtypeStruct((8, 128), jnp.int32),
        ],
        scratch_shapes=[pltpu.VMEM((8, 128), jnp.float32)],
    )(x, gate_w)


def _dispatch_kernel(x_hbm, s0_hbm, s1_hbm, xs_hbm, xv, i0v, i1v, sem):
    wid = lax.axis_index("s") * 2 + lax.axis_index("c")
    pltpu.sync_copy(s0_hbm.at[wid], i0v)
    pltpu.sync_copy(s1_hbm.at[wid], i1v)
    for j in range(_NCH):
        base = wid * _ROWS_W + j * _CH
        pltpu.sync_copy(x_hbm.at[pl.ds(base, _CH), :], xv)
        pltpu.async_copy(xv, xs_hbm.at[i0v.at[j]], sem).wait()
        pltpu.async_copy(xv, xs_hbm.at[i1v.at[j]], sem).wait()


def _run_dispatch(x, s0r, s1r):
    mesh = plsc.VectorSubcoreMesh(core_axis_name="c", subcore_axis_name="s")
    return pl.kernel(
        _dispatch_kernel,
        out_type=jax.ShapeDtypeStruct((_TKP, _D), jnp.float32),
        mesh=mesh,
        scratch_types=[
            pltpu.VMEM((_CH, _D), jnp.float32),
            pltpu.VMEM((_NCH, _CH), jnp.int32),
            pltpu.VMEM((_NCH, _CH), jnp.int32),
            pltpu.SemaphoreType.DMA,
        ],
    )(x, s0r, s1r)


def _gmm_kernel(e_ref, u_ref, xs_ref, wg_ref, wu_ref, wd_ref, ys_ref, xb_ref):
    f = pl.program_id(1)
    used = u_ref[pl.program_id(0)]
    nsub = (used + _SUB - 1) // _SUB
    wg = wg_ref[0].astype(jnp.bfloat16)
    wu = wu_ref[0].astype(jnp.bfloat16)
    wd = wd_ref[0].astype(jnp.bfloat16)

    def _case(k):
        sls = [pl.ds(j * _SUB, _SUB) for j in range(k)]

        @pl.when(f == 0)
        def _():
            for sl in sls:
                xb_ref[sl, :] = xs_ref[sl, :].astype(jnp.bfloat16)

        for p in range(0, k, 2):
            grp = sls[p:p + 2]
            xsubs = [xb_ref[sl, :] for sl in grp]
            gs = [jnp.dot(x, wg, preferred_element_type=jnp.float32)
                  for x in xsubs]
            us = [jnp.dot(x, wu, preferred_element_type=jnp.float32)
                  for x in xsubs]
            hs = [(g * lax.logistic(g) * u).astype(jnp.bfloat16)
                  for g, u in zip(gs, us)]
            for sl, h in zip(grp, hs):
                y = jnp.dot(h, wd, preferred_element_type=jnp.float32)
                prev = jnp.where(f == 0, 0.0, ys_ref[sl, :])
                ys_ref[sl, :] = prev + y

    for k in range(1, _NSUB + 1):
        @pl.when(nsub == k)
        def _(k=k):
            _case(k)


def _run_gmm(xs, w_gate, w_up, w_down, e_of_tile, used):
    grid_spec = pltpu.PrefetchScalarGridSpec(
        num_scalar_prefetch=2,
        grid=(_NT, _NF),
        in_specs=[
            pl.BlockSpec((_SG, _D),
                         lambda i, f, e, u: (jnp.where(u[i] > 0, i, 0), 0)),
            pl.BlockSpec((1, _D, _BF),
                         lambda i, f, e, u: (e[i], 0, jnp.where(u[i] > 0, f, 0))),
            pl.BlockSpec((1, _D, _BF),
                         lambda i, f, e, u: (e[i], 0, jnp.where(u[i] > 0, f, 0))),
            pl.BlockSpec((1, _BF, _D),
                         lambda i, f, e, u: (e[i], jnp.where(u[i] > 0, f, 0), 0)),
        ],
        out_specs=pl.BlockSpec((_SG, _D), lambda i, f, e, u: (i, 0)),
        scratch_shapes=[pltpu.VMEM((_SG, _D), jnp.bfloat16)],
    )
    return pl.pallas_call(
        _gmm_kernel,
        grid_spec=grid_spec,
        out_shape=jax.ShapeDtypeStruct((_TKP, _D), jnp.float32),
    )(e_of_tile, used, xs, w_gate, w_up, w_down)


def _collect_kernel(ys_hbm, s0_hbm, s1_hbm, y0_hbm, y1_hbm, rv, i0v, i1v, sem):
    wid = lax.axis_index("s") * 2 + lax.axis_index("c")
    pltpu.sync_copy(s0_hbm.at[wid], i0v)
    pltpu.sync_copy(s1_hbm.at[wid], i1v)
    for j in range(_NCH):
        base = wid * _ROWS_W + j * _CH
        pltpu.async_copy(ys_hbm.at[i0v.at[j]], rv, sem).wait()
        pltpu.sync_copy(rv, y0_hbm.at[pl.ds(base, _CH), :])
        pltpu.async_copy(ys_hbm.at[i1v.at[j]], rv, sem).wait()
        pltpu.sync_copy(rv, y1_hbm.at[pl.ds(base, _CH), :])


def _run_collect(ys, s0r, s1r):
    mesh = plsc.VectorSubcoreMesh(core_axis_name="c", subcore_axis_name="s")
    return pl.kernel(
        _collect_kernel,
        out_type=(
            jax.ShapeDtypeStruct((_T, _D), jnp.float32),
            jax.ShapeDtypeStruct((_T, _D), jnp.float32),
        ),
        mesh=mesh,
        scratch_types=[
            pltpu.VMEM((_CH, _D), jnp.float32),
            pltpu.VMEM((_NCH, _CH), jnp.int32),
            pltpu.VMEM((_NCH, _CH), jnp.int32),
            pltpu.SemaphoreType.DMA,
        ],
    )(ys, s0r, s1r)


def _combine_kernel(y0_ref, y1_ref, w_ref, out_ref):
    w = w_ref[...]
    out_ref[...] = y0_ref[...] * w[:, 0:1] + y1_ref[...] * w[:, 1:2]


def _run_combine(y0, y1, flts):
    return pl.pallas_call(
        _combine_kernel,
        grid=(_NRB,),
        in_specs=[
            pl.BlockSpec((_RB, _D), lambda i: (i, 0)),
            pl.BlockSpec((_RB, _D), lambda i: (i, 0)),
            pl.BlockSpec((_RB, 128), lambda i: (i, 0)),
        ],
        out_specs=pl.BlockSpec((_RB, _D), lambda i: (i, 0)),
        out_shape=jax.ShapeDtypeStruct((_T, _D), jnp.float32),
    )(y0, y1, flts)


def kernel(hidden_states, gate_w, w_gate, w_up, w_down):
    b, s, d = hidden_states.shape
    x = hidden_states.reshape(-1, d)

    ints, flts, cnts = _run_router(x, gate_w)
    i0 = ints[:, 0]
    i1 = ints[:, 1]
    p0 = ints[:, 2]
    p1 = ints[:, 3]
    counts = cnts[0, :_E]

    sizes_pad = ((counts + _SG - 1) // _SG) * _SG
    ends = jnp.cumsum(sizes_pad)
    offs = ends - sizes_pad
    slot0 = (offs[i0] + p0).astype(jnp.int32)
    slot1 = (offs[i1] + p1).astype(jnp.int32)
    s0r = slot0.reshape(_NW, _NCH, _CH)
    s1r = slot1.reshape(_NW, _NCH, _CH)
    tile_start = jnp.arange(_NT, dtype=jnp.int32) * _SG
    e_of_tile = jnp.minimum(
        jnp.searchsorted(ends, tile_start, side="right"), _E - 1
    ).astype(jnp.int32)
    used = jnp.clip(counts[e_of_tile] - (tile_start - offs[e_of_tile]),
                    0, _SG).astype(jnp.int32)

    xs = _run_dispatch(x, s0r, s1r)
    ys = _run_gmm(xs, w_gate, w_up, w_down, e_of_tile, used)
    y0, y1 = _run_collect(ys, s0r, s1r)
    out = _run_combine(y0, y1, flts)
    return out.reshape(b, s, d)
